# baseline (speedup 1.0000x reference)
import functools

import jax
import jax.numpy as jnp
from jax import lax
from jax.experimental import pallas as pl
from jax.experimental.pallas import tpu as pltpu
from jax.experimental.pallas import tpu_sc as plsc

DIM = 128
L = 16
NC = 2
NS = 16
NW = NC * NS
CHUNK = 128
TBLK = 2048


def _gather_partial_body(tw_hbm, cw_hbm, tt_hbm, ct_hbm, out_hbm,
                         tw_v, cw_v, a_v, b_v, part_v, sem_a, sem_b):
    nchunk = tw_v.shape[0]
    wid = lax.axis_index("s") * NC + lax.axis_index("c")
    pltpu.sync_copy(tw_hbm.at[wid], tw_v)
    pltpu.sync_copy(cw_hbm.at[wid], cw_v)

    @pl.loop(0, nchunk)
    def _chunk(c):
        ca = pltpu.async_copy(tt_hbm.at[tw_v.at[c]], a_v, sem_a)
        cb = pltpu.async_copy(ct_hbm.at[cw_v.at[c]], b_v, sem_b)
        ca.wait()
        cb.wait()
        for r in range(CHUNK):
            acc = a_v[r, pl.ds(0, L)] * b_v[r, pl.ds(0, L)]
            for k in range(1, DIM // L):
                acc = acc + a_v[r, pl.ds(k * L, L)] * b_v[r, pl.ds(k * L, L)]
            part_v[c, r] = acc

    pltpu.sync_copy(part_v, out_hbm.at[wid])


def _reduce_body(p_ref, o_ref):
    x = p_ref[...]
    s = jnp.sum(x.reshape(x.shape[0], L), axis=1)
    o_ref[...] = jax.nn.sigmoid(s)


@jax.jit
def _skipgram(target_word, context_word, target_table, context_table):
    batch = target_word.shape[0]
    b_per_w = batch // NW
    nchunk = b_per_w // CHUNK
    tw3 = target_word.reshape(NW, nchunk, CHUNK).astype(jnp.int32)
    cw3 = context_word.reshape(NW, nchunk, CHUNK).astype(jnp.int32)

    mesh = plsc.VectorSubcoreMesh(core_axis_name="c", subcore_axis_name="s")
    gather_partial = pl.kernel(
        _gather_partial_body,
        out_type=jax.ShapeDtypeStruct((NW, nchunk, CHUNK, L), jnp.float32),
        mesh=mesh,
        scratch_types=[
            pltpu.VMEM((nchunk, CHUNK), jnp.int32),
            pltpu.VMEM((nchunk, CHUNK), jnp.int32),
            pltpu.VMEM((CHUNK, DIM), jnp.float32),
            pltpu.VMEM((CHUNK, DIM), jnp.float32),
            pltpu.VMEM((nchunk, CHUNK, L), jnp.float32),
            pltpu.SemaphoreType.DMA,
            pltpu.SemaphoreType.DMA,
        ],
    )
    partial = gather_partial(tw3, cw3, target_table, context_table)
    partial = partial.reshape(batch, L)

    out = pl.pallas_call(
        _reduce_body,
        grid=(batch // TBLK,),
        in_specs=[pl.BlockSpec((TBLK, L), lambda i: (i, 0))],
        out_specs=pl.BlockSpec((TBLK,), lambda i: (i,)),
        out_shape=jax.ShapeDtypeStruct((batch,), jnp.float32),
    )(partial)
    return out


def kernel(target_word, context_word, target_table, context_table):
    return _skipgram(target_word, context_word, target_table, context_table)


# speedup vs baseline: 1.4659x; 1.4659x over previous
import functools

import jax
import jax.numpy as jnp
from jax import lax
from jax.experimental import pallas as pl
from jax.experimental.pallas import tpu as pltpu
from jax.experimental.pallas import tpu_sc as plsc

DIM = 128
L = 16
NC = 2
NS = 16
NW = NC * NS
CHUNK = 128


def _gather_partial_body(tw_hbm, cw_hbm, tt_hbm, ct_hbm, out_hbm,
                         tw_v, cw_v, a0, b0, a1, b1, part_v,
                         sa0, sb0, sa1, sb1):
    nchunk = tw_v.shape[0]
    wid = lax.axis_index("s") * NC + lax.axis_index("c")
    pltpu.sync_copy(tw_hbm.at[wid], tw_v)
    pltpu.sync_copy(cw_hbm.at[wid], cw_v)

    bufs = ((a0, b0, sa0, sb0), (a1, b1, sa1, sb1))
    descs = {}

    def issue(c):
        a_v, b_v, sa, sb = bufs[c % 2]
        descs[c] = (pltpu.async_copy(tt_hbm.at[tw_v.at[c]], a_v, sa),
                    pltpu.async_copy(ct_hbm.at[cw_v.at[c]], b_v, sb))

    issue(0)
    for c in range(nchunk):
        if c + 1 < nchunk:
            issue(c + 1)
        da, db = descs.pop(c)
        da.wait()
        db.wait()
        a_v, b_v, _, _ = bufs[c % 2]

        @pl.loop(0, CHUNK, unroll=8)
        def _row(r):
            acc = a_v[r, pl.ds(0, L)] * b_v[r, pl.ds(0, L)]
            for k in range(1, DIM // L):
                acc = acc + a_v[r, pl.ds(k * L, L)] * b_v[r, pl.ds(k * L, L)]
            part_v[pl.ds((c * CHUNK + r) * L, L)] = acc

    pltpu.sync_copy(part_v, out_hbm.at[wid])


def _reduce_body(p_ref, m_ref, o_ref):
    s = jnp.dot(p_ref[...], m_ref[...], preferred_element_type=jnp.float32)
    o_ref[...] = jax.nn.sigmoid(s)


@jax.jit
def _skipgram(target_word, context_word, target_table, context_table):
    batch = target_word.shape[0]
    b_per_w = batch // NW
    nchunk = b_per_w // CHUNK
    tw3 = target_word.reshape(NW, nchunk, CHUNK).astype(jnp.int32)
    cw3 = context_word.reshape(NW, nchunk, CHUNK).astype(jnp.int32)

    mesh = plsc.VectorSubcoreMesh(core_axis_name="c", subcore_axis_name="s")
    gather_partial = pl.kernel(
        _gather_partial_body,
        out_type=jax.ShapeDtypeStruct((NW, nchunk * CHUNK * L), jnp.float32),
        mesh=mesh,
        scratch_types=[
            pltpu.VMEM((nchunk, CHUNK), jnp.int32),
            pltpu.VMEM((nchunk, CHUNK), jnp.int32),
            pltpu.VMEM((CHUNK, DIM), jnp.float32),
            pltpu.VMEM((CHUNK, DIM), jnp.float32),
            pltpu.VMEM((CHUNK, DIM), jnp.float32),
            pltpu.VMEM((CHUNK, DIM), jnp.float32),
            pltpu.VMEM((nchunk * CHUNK * L,), jnp.float32),
            pltpu.SemaphoreType.DMA,
            pltpu.SemaphoreType.DMA,
            pltpu.SemaphoreType.DMA,
            pltpu.SemaphoreType.DMA,
        ],
    )
    partial = gather_partial(tw3, cw3, target_table, context_table)
    partial = partial.reshape(batch // 8, 8 * L)

    seg = jnp.repeat(jnp.eye(8, dtype=jnp.float32), L, axis=0)

    out = pl.pallas_call(
        _reduce_body,
        out_shape=jax.ShapeDtypeStruct((batch // 8, 8), jnp.float32),
    )(partial, seg)
    return out.reshape(batch)


def kernel(target_word, context_word, target_table, context_table):
    return _skipgram(target_word, context_word, target_table, context_table)


# speedup vs baseline: 1.6896x; 1.1527x over previous
import functools

import numpy as np
import jax
import jax.numpy as jnp
from jax import lax
from jax.experimental import pallas as pl
from jax.experimental.pallas import tpu as pltpu
from jax.experimental.pallas import tpu_sc as plsc

DIM = 128
L = 16
NC = 2
NS = 16
NW = NC * NS
CHUNK = 128

_SEG = np.kron(np.eye(DIM, dtype=np.float32), np.ones((L, 1), np.float32))


def _gather_partial_body(tw_hbm, cw_hbm, tt_hbm, ct_hbm, out_hbm,
                         tw_v, cw_v, a0, b0, a1, b1, part_v,
                         sa0, sb0, sa1, sb1):
    b_per_w = tw_v.shape[0]
    nchunk = b_per_w // CHUNK
    wid = lax.axis_index("s") * NC + lax.axis_index("c")
    pltpu.sync_copy(tw_hbm.at[wid], tw_v)
    pltpu.sync_copy(cw_hbm.at[wid], cw_v)

    bufs = ((a0, b0, sa0, sb0), (a1, b1, sa1, sb1))
    descs = {}

    def issue(c):
        a_v, b_v, sa, sb = bufs[c % 2]
        idx_t = tw_v.at[pl.ds(c * CHUNK, CHUNK)]
        idx_c = cw_v.at[pl.ds(c * CHUNK, CHUNK)]
        descs[c] = (pltpu.async_copy(tt_hbm.at[idx_t], a_v, sa),
                    pltpu.async_copy(ct_hbm.at[idx_c], b_v, sb))

    issue(0)
    for c in range(nchunk):
        if c + 1 < nchunk:
            issue(c + 1)
        da, db = descs.pop(c)
        da.wait()
        db.wait()
        a_v, b_v, _, _ = bufs[c % 2]

        @pl.loop(0, CHUNK, unroll=8)
        def _row(r):
            acc = a_v[r, pl.ds(0, L)] * b_v[r, pl.ds(0, L)]
            for k in range(1, DIM // L):
                acc = acc + a_v[r, pl.ds(k * L, L)] * b_v[r, pl.ds(k * L, L)]
            part_v[c, pl.ds(r * L, L)] = acc

    pltpu.sync_copy(part_v, out_hbm.at[pl.ds(wid * nchunk, nchunk)])


def _reduce_body(p_ref, m_ref, o_ref):
    s = jnp.dot(p_ref[...], m_ref[...], preferred_element_type=jnp.float32)
    o_ref[...] = jax.nn.sigmoid(s)


@jax.jit
def _skipgram(target_word, context_word, target_table, context_table):
    batch = target_word.shape[0]
    b_per_w = batch // NW
    nchunk = b_per_w // CHUNK
    tw2 = target_word.reshape(NW, b_per_w).astype(jnp.int32)
    cw2 = context_word.reshape(NW, b_per_w).astype(jnp.int32)

    mesh = plsc.VectorSubcoreMesh(core_axis_name="c", subcore_axis_name="s")
    gather_partial = pl.kernel(
        _gather_partial_body,
        out_type=jax.ShapeDtypeStruct((NW * nchunk, CHUNK * L), jnp.float32),
        mesh=mesh,
        scratch_types=[
            pltpu.VMEM((b_per_w,), jnp.int32),
            pltpu.VMEM((b_per_w,), jnp.int32),
            pltpu.VMEM((CHUNK, DIM), jnp.float32),
            pltpu.VMEM((CHUNK, DIM), jnp.float32),
            pltpu.VMEM((CHUNK, DIM), jnp.float32),
            pltpu.VMEM((CHUNK, DIM), jnp.float32),
            pltpu.VMEM((nchunk, CHUNK * L), jnp.float32),
            pltpu.SemaphoreType.DMA,
            pltpu.SemaphoreType.DMA,
            pltpu.SemaphoreType.DMA,
            pltpu.SemaphoreType.DMA,
        ],
    )
    partial = gather_partial(tw2, cw2, target_table, context_table)

    out = pl.pallas_call(
        _reduce_body,
        out_shape=jax.ShapeDtypeStruct((NW * nchunk, CHUNK), jnp.float32),
    )(partial, jnp.asarray(_SEG))
    return out.reshape(batch)


def kernel(target_word, context_word, target_table, context_table):
    return _skipgram(target_word, context_word, target_table, context_table)


# speedup vs baseline: 1.6975x; 1.0046x over previous
import functools

import numpy as np
import jax
import jax.numpy as jnp
from jax import lax
from jax.experimental import pallas as pl
from jax.experimental.pallas import tpu as pltpu
from jax.experimental.pallas import tpu_sc as plsc

DIM = 128
L = 16
NC = 2
NS = 16
NW = NC * NS
CHUNK = 128

_SEG = np.kron(np.eye(DIM, dtype=np.float32), np.ones((L, 1), np.float32))


def _gather_partial_body(tw_hbm, cw_hbm, tt_hbm, ct_hbm, out_hbm,
                         tw_v, cw_v, a0, b0, a1, b1, part_v,
                         sa0, sb0, sa1, sb1):
    b_per_w = tw_v.shape[0]
    nchunk = b_per_w // CHUNK
    wid = lax.axis_index("s") * NC + lax.axis_index("c")
    pltpu.sync_copy(tw_hbm.at[pl.ds(wid * b_per_w, b_per_w)], tw_v)
    pltpu.sync_copy(cw_hbm.at[pl.ds(wid * b_per_w, b_per_w)], cw_v)

    bufs = ((a0, b0, sa0, sb0), (a1, b1, sa1, sb1))
    descs = {}

    def issue(c):
        a_v, b_v, sa, sb = bufs[c % 2]
        idx_t = tw_v.at[pl.ds(c * CHUNK, CHUNK)]
        idx_c = cw_v.at[pl.ds(c * CHUNK, CHUNK)]
        descs[c] = (pltpu.async_copy(tt_hbm.at[idx_t], a_v, sa),
                    pltpu.async_copy(ct_hbm.at[idx_c], b_v, sb))

    issue(0)
    for c in range(nchunk):
        if c + 1 < nchunk:
            issue(c + 1)
        da, db = descs.pop(c)
        da.wait()
        db.wait()
        a_v, b_v, _, _ = bufs[c % 2]

        @pl.loop(0, CHUNK, unroll=8)
        def _row(r):
            acc = a_v[r, pl.ds(0, L)] * b_v[r, pl.ds(0, L)]
            for k in range(1, DIM // L):
                acc = acc + a_v[r, pl.ds(k * L, L)] * b_v[r, pl.ds(k * L, L)]
            part_v[c, pl.ds(r * L, L)] = acc

    pltpu.sync_copy(part_v, out_hbm.at[pl.ds(wid * nchunk, nchunk)])


def _reduce_body(p_ref, m_ref, o_ref):
    s = jnp.dot(p_ref[...], m_ref[...], preferred_element_type=jnp.float32)
    o_ref[...] = jax.nn.sigmoid(s)


@jax.jit
def _skipgram(target_word, context_word, target_table, context_table):
    batch = target_word.shape[0]
    b_per_w = batch // NW
    nchunk = b_per_w // CHUNK
    tw1 = target_word.astype(jnp.int32)
    cw1 = context_word.astype(jnp.int32)

    mesh = plsc.VectorSubcoreMesh(core_axis_name="c", subcore_axis_name="s")
    gather_partial = pl.kernel(
        _gather_partial_body,
        out_type=jax.ShapeDtypeStruct((NW * nchunk, CHUNK * L), jnp.float32),
        mesh=mesh,
        scratch_types=[
            pltpu.VMEM((b_per_w,), jnp.int32),
            pltpu.VMEM((b_per_w,), jnp.int32),
            pltpu.VMEM((CHUNK, DIM), jnp.float32),
            pltpu.VMEM((CHUNK, DIM), jnp.float32),
            pltpu.VMEM((CHUNK, DIM), jnp.float32),
            pltpu.VMEM((CHUNK, DIM), jnp.float32),
            pltpu.VMEM((nchunk, CHUNK * L), jnp.float32),
            pltpu.SemaphoreType.DMA,
            pltpu.SemaphoreType.DMA,
            pltpu.SemaphoreType.DMA,
            pltpu.SemaphoreType.DMA,
        ],
    )
    partial = gather_partial(tw1, cw1, target_table, context_table)

    out = pl.pallas_call(
        _reduce_body,
        out_shape=jax.ShapeDtypeStruct((NW * nchunk, CHUNK), jnp.float32),
    )(partial, jnp.asarray(_SEG))
    return out.reshape(batch)


def kernel(target_word, context_word, target_table, context_table):
    return _skipgram(target_word, context_word, target_table, context_table)


# speedup vs baseline: 1.7910x; 1.0551x over previous
import functools

import numpy as np
import jax
import jax.numpy as jnp
from jax import lax
from jax.experimental import pallas as pl
from jax.experimental.pallas import tpu as pltpu
from jax.experimental.pallas import tpu_sc as plsc

DIM = 128
L = 16
NC = 2
NS = 16
NW = NC * NS
CHUNK = 128

_SEG = np.kron(np.eye(DIM, dtype=np.float32), np.ones((L, 1), np.float32))


def _gather_partial_body(tw_hbm, cw_hbm, tt_hbm, ct_hbm, out_hbm,
                         tw_v, cw_v, a0, b0, a1, b1, part_v,
                         sa0, sb0, sa1, sb1):
    b_per_w = tw_v.shape[0]
    nchunk = b_per_w // CHUNK
    wid = lax.axis_index("s") * NC + lax.axis_index("c")
    pltpu.sync_copy(tw_hbm.at[pl.ds(wid * b_per_w, b_per_w)], tw_v)
    pltpu.sync_copy(cw_hbm.at[pl.ds(wid * b_per_w, b_per_w)], cw_v)

    bufs = ((a0, b0, sa0, sb0), (a1, b1, sa1, sb1))
    descs = {}

    def issue(c):
        a_v, b_v, sa, sb = bufs[c % 2]
        idx_t = tw_v.at[pl.ds(c * CHUNK, CHUNK)]
        idx_c = cw_v.at[pl.ds(c * CHUNK, CHUNK)]
        descs[c] = (pltpu.async_copy(tt_hbm.at[idx_t], a_v, sa),
                    pltpu.async_copy(ct_hbm.at[idx_c], b_v, sb))

    issue(0)
    for c in range(nchunk):
        if c + 1 < nchunk:
            issue(c + 1)
        da, db = descs.pop(c)
        da.wait()
        db.wait()
        a_v, b_v, _, _ = bufs[c % 2]

        @plsc.parallel_loop(0, CHUNK, unroll=8)
        def _row(r):
            acc = a_v[r, pl.ds(0, L)] * b_v[r, pl.ds(0, L)]
            for k in range(1, DIM // L):
                acc = acc + a_v[r, pl.ds(k * L, L)] * b_v[r, pl.ds(k * L, L)]
            part_v[c, pl.ds(r * L, L)] = acc

    pltpu.sync_copy(part_v, out_hbm.at[pl.ds(wid * nchunk, nchunk)])


def _reduce_body(p_ref, m_ref, o_ref):
    s = jnp.dot(p_ref[...], m_ref[...], preferred_element_type=jnp.float32)
    o_ref[...] = jax.nn.sigmoid(s)


@jax.jit
def _skipgram(target_word, context_word, target_table, context_table):
    batch = target_word.shape[0]
    b_per_w = batch // NW
    nchunk = b_per_w // CHUNK
    tw1 = target_word.astype(jnp.int32)
    cw1 = context_word.astype(jnp.int32)

    mesh = plsc.VectorSubcoreMesh(core_axis_name="c", subcore_axis_name="s")
    gather_partial = pl.kernel(
        _gather_partial_body,
        out_type=jax.ShapeDtypeStruct((NW * nchunk, CHUNK * L), jnp.float32),
        mesh=mesh,
        scratch_types=[
            pltpu.VMEM((b_per_w,), jnp.int32),
            pltpu.VMEM((b_per_w,), jnp.int32),
            pltpu.VMEM((CHUNK, DIM), jnp.float32),
            pltpu.VMEM((CHUNK, DIM), jnp.float32),
            pltpu.VMEM((CHUNK, DIM), jnp.float32),
            pltpu.VMEM((CHUNK, DIM), jnp.float32),
            pltpu.VMEM((nchunk, CHUNK * L), jnp.float32),
            pltpu.SemaphoreType.DMA,
            pltpu.SemaphoreType.DMA,
            pltpu.SemaphoreType.DMA,
            pltpu.SemaphoreType.DMA,
        ],
    )
    partial = gather_partial(tw1, cw1, target_table, context_table)

    out = pl.pallas_call(
        _reduce_body,
        out_shape=jax.ShapeDtypeStruct((NW * nchunk, CHUNK), jnp.float32),
    )(partial, jnp.asarray(_SEG))
    return out.reshape(batch)


def kernel(target_word, context_word, target_table, context_table):
    return _skipgram(target_word, context_word, target_table, context_table)
